# Initial kernel scaffold; baseline (speedup 1.0000x reference)
import jax
import jax.numpy as jnp
from jax.experimental import pallas as pl


def kernel(fushed_features, input_data, centroids, gcn_w, bn_gamma, bn_beta, adj_mx_topk_index):
    raise NotImplementedError("write your pallas kernel here")


# speedup vs baseline: 3.4680x; 3.4680x over previous
import functools

import jax
import jax.numpy as jnp
from jax import lax
from jax.experimental import pallas as pl
from jax.experimental.pallas import tpu as pltpu
from jax.experimental.pallas import tpu_sc as plsc

B, S, D, C, K, T, TO = 64, 1024, 256, 8, 16, 12, 12
MARGIN = 1.0
CT = C * TO
NEG = -1e30


def _stats_body(x_ref, out_ref):
    b = pl.program_id(0)
    x = x_ref[0]
    s1 = jnp.sum(x, axis=1, keepdims=True)
    s2 = jnp.sum(x * x, axis=1, keepdims=True)
    cur = jnp.concatenate([s1, s2], axis=1)

    @pl.when(b == 0)
    def _():
        out_ref[...] = jnp.zeros_like(out_ref)

    out_ref[...] += cur


def _stats(x):
    return pl.pallas_call(
        _stats_body,
        grid=(B,),
        in_specs=[pl.BlockSpec((1, S, D), lambda b: (b, 0, 0))],
        out_specs=pl.BlockSpec((S, 2), lambda b: (0, 0)),
        out_shape=jax.ShapeDtypeStruct((S, 2), jnp.float32),
    )(x)


def _dense_body(x_ref, sq_ref, st_ref, g_ref, be_ref, cent_ref, w_ref,
                idx_ref, q_ref, sp_ref, gidx_ref):
    b = pl.program_id(0)
    st = st_ref[...]
    inv = 1.0 / (B * D)
    mean = st[:, 0:1] * inv
    var = st[:, 1:2] * inv - mean * mean
    a = g_ref[...] * jax.lax.rsqrt(var + 1e-5)
    bv = be_ref[...] - mean * a

    fn = x_ref[0] * a + bv
    adj = jnp.mean(fn, axis=0, keepdims=True)
    x1 = fn - adj
    x2 = cent_ref[...] - adj
    x1n = jnp.sum(x1 * x1, axis=1, keepdims=True)
    x2n = jnp.reshape(jnp.sum(x2 * x2, axis=1), (1, C))
    cross = jax.lax.dot_general(x1, x2, (((1,), (1,)), ((), ())),
                                preferred_element_type=jnp.float32)
    simi = jnp.sqrt(jnp.clip(x1n + x2n - 2.0 * cross, 1e-30, None))
    m = jnp.max(simi, axis=1, keepdims=True)
    e = jnp.exp(simi - m)
    sa = e / jnp.sum(e, axis=1, keepdims=True)

    h = jax.lax.dot_general(sq_ref[0], w_ref[...], (((1,), (1,)), ((), ())),
                            preferred_element_type=jnp.float32)
    h = jnp.maximum(h, 0.0)
    rep = jnp.concatenate(
        [jnp.broadcast_to(sa[:, c:c + 1], (S, TO)) for c in range(C)], axis=1)
    q_ref[0] = h * rep

    sp_ref[0] = jnp.concatenate(
        [sa, jnp.full((S, C), NEG, jnp.float32)], axis=1)
    gidx_ref[0] = idx_ref[0] + b * S


def _dense(x, sq, st, gamma, beta, cent, w96, idx):
    return pl.pallas_call(
        _dense_body,
        grid=(B,),
        in_specs=[
            pl.BlockSpec((1, S, D), lambda b: (b, 0, 0)),
            pl.BlockSpec((1, S, T), lambda b: (b, 0, 0)),
            pl.BlockSpec((S, 2), lambda b: (0, 0)),
            pl.BlockSpec((S, 1), lambda b: (0, 0)),
            pl.BlockSpec((S, 1), lambda b: (0, 0)),
            pl.BlockSpec((C, D), lambda b: (0, 0)),
            pl.BlockSpec((CT, T), lambda b: (0, 0)),
            pl.BlockSpec((1, S, K), lambda b: (b, 0, 0)),
        ],
        out_specs=[
            pl.BlockSpec((1, S, CT), lambda b: (b, 0, 0)),
            pl.BlockSpec((1, S, 2 * C), lambda b: (b, 0, 0)),
            pl.BlockSpec((1, S, K), lambda b: (b, 0, 0)),
        ],
        out_shape=[
            jax.ShapeDtypeStruct((B, S, CT), jnp.float32),
            jax.ShapeDtypeStruct((B, S, 2 * C), jnp.float32),
            jax.ShapeDtypeStruct((B, S, K), jnp.int32),
        ],
    )(x, sq, st, gamma, beta, cent, w96, idx)


_NTOT = B * S * K
_ROWW = 2 * C


def _make_sc_gather():
    info = plsc.get_sparse_core_info()
    nw = info.num_cores * info.num_subcores
    per_w = _NTOT // nw
    ch = 4096
    n_ch = per_w // ch
    mesh = plsc.VectorSubcoreMesh(core_axis_name="c", subcore_axis_name="s")

    @functools.partial(
        pl.kernel, mesh=mesh,
        compiler_params=pltpu.CompilerParams(use_tc_tiling_on_sc=False),
        out_type=jax.ShapeDtypeStruct((_NTOT, _ROWW), jnp.float32),
        scratch_types=[
            pltpu.VMEM((ch,), jnp.int32),
            pltpu.VMEM((ch, _ROWW), jnp.float32),
            pltpu.SemaphoreType.DMA,
        ],
    )
    def gather_k(table_hbm, gidx_hbm, out_hbm, idx_v, rows_v, sem):
        wid = lax.axis_index("s") * info.num_cores + lax.axis_index("c")
        base = wid * per_w
        for i in range(n_ch):
            off = base + i * ch
            pltpu.sync_copy(gidx_hbm.at[pl.ds(off, ch)], idx_v)
            pltpu.async_copy(table_hbm.at[idx_v], rows_v, sem).wait()
            pltpu.sync_copy(rows_v, out_hbm.at[pl.ds(off, ch)])

    return gather_k


def _sc_gather(table, gidx):
    return _make_sc_gather()(table, gidx)


def _reduce_body(q_ref, idx_ref, r_ref, upd_ref, cnt_ref, a1_ref, l3_ref):
    b = pl.program_id(0)

    @pl.when(b == 0)
    def _():
        cnt_ref[...] = jnp.zeros_like(cnt_ref)
        a1_ref[...] = jnp.zeros_like(a1_ref)
        l3_ref[...] = jnp.zeros_like(l3_ref)

    q = q_ref[0]
    CH = 128
    l3_tot = jnp.zeros((1, 1), jnp.float32)
    for r0 in range(0, S, CH):
        idx = idx_ref[0, r0:r0 + CH, :]
        jiota = jax.lax.broadcasted_iota(jnp.int32, (CH, S), 1)
        w = jnp.where(idx[:, 0:1] == jiota, 1.0, 0.0)
        for k in range(1, K):
            w = w + jnp.where(idx[:, k:k + 1] == jiota, 1.0, 0.0)
        u = jax.lax.dot_general(w, q, (((1,), (0,)), ((), ())),
                                preferred_element_type=jnp.float32)

        r2 = r_ref[0, r0:r0 + CH, :]
        rk = [r2[:, k * _ROWW:k * _ROWW + C] for k in range(K)]
        den = rk[0]
        for k in range(1, K):
            den = den + rk[k]
        deng = jnp.where(den == 0.0, 1.0, den)
        upd_ref[0, r0:r0 + CH, :] = u / jnp.concatenate(
            [jnp.broadcast_to(deng[:, c:c + 1], (CH, TO)) for c in range(C)],
            axis=1)

        lane = jax.lax.broadcasted_iota(jnp.int32, (CH, C), 1)
        ohs, lps = [], []
        for k in range(K):
            x = rk[k]
            mx = jnp.max(x, axis=1, keepdims=True)
            cand = jnp.where(x == mx, lane, C + 1)
            am = jnp.min(cand, axis=1, keepdims=True)
            ohs.append(jnp.where(lane == am, 1.0, 0.0))
            lse = mx + jnp.log(jnp.sum(jnp.exp(x - mx), axis=1, keepdims=True))
            lps.append(x - lse)
        cnt_ref[r0:r0 + CH, :] += jnp.concatenate(ohs, axis=1)
        a1_ref[r0:r0 + CH, :] += jnp.concatenate(lps, axis=1)

        mu = den * (1.0 / K)
        xk = [r - mu for r in rk]
        nk = [jnp.sum(x * x, axis=1, keepdims=True) for x in xk]
        acc = jnp.zeros((CH, 1), jnp.float32)
        for k in range(K):
            for l in range(k + 1, K):
                dot = jnp.sum(xk[k] * xk[l], axis=1, keepdims=True)
                d = jnp.sqrt(jnp.clip(nk[k] + nk[l] - 2.0 * dot, 1e-30, None))
                c = jnp.clip(MARGIN - d, 0.0, None)
                acc = acc + c * c
        l3_tot = l3_tot + jnp.reshape(jnp.sum(acc), (1, 1))
    l3_ref[...] += l3_tot * (2.0 / (B * S))


def _reduce(q, idx, r2):
    return pl.pallas_call(
        _reduce_body,
        grid=(B,),
        in_specs=[
            pl.BlockSpec((1, S, CT), lambda b: (b, 0, 0)),
            pl.BlockSpec((1, S, K), lambda b: (b, 0, 0)),
            pl.BlockSpec((1, S, K * _ROWW), lambda b: (b, 0, 0)),
        ],
        out_specs=[
            pl.BlockSpec((1, S, CT), lambda b: (b, 0, 0)),
            pl.BlockSpec((S, K * C), lambda b: (0, 0)),
            pl.BlockSpec((S, K * C), lambda b: (0, 0)),
            pl.BlockSpec((1, 1), lambda b: (0, 0)),
        ],
        out_shape=[
            jax.ShapeDtypeStruct((B, S, CT), jnp.float32),
            jax.ShapeDtypeStruct((S, K * C), jnp.float32),
            jax.ShapeDtypeStruct((S, K * C), jnp.float32),
            jax.ShapeDtypeStruct((1, 1), jnp.float32),
        ],
    )(q, idx, r2)


def _final_body(cnt_ref, a1_ref, cent_ref, l1_ref, l2_ref):
    t1 = jnp.zeros((S, 1), jnp.float32)
    t2 = jnp.zeros((S, 1), jnp.float32)
    for k in range(K):
        cn = cnt_ref[:, k * C:(k + 1) * C] * (1.0 / B)
        m = jnp.max(cn, axis=1, keepdims=True)
        e = jnp.exp(cn - m)
        gt = e / jnp.sum(e, axis=1, keepdims=True)
        t1 = t1 + jnp.sum(gt * jnp.log(gt), axis=1, keepdims=True)
        t2 = t2 + jnp.sum(gt * a1_ref[:, k * C:(k + 1) * C], axis=1,
                          keepdims=True)
    l1_ref[...] = jnp.reshape(
        jnp.sum(t1) * (1.0 / S) - jnp.sum(t2) * (1.0 / (B * S)), (1, 1))

    cent = cent_ref[...]
    adj = jnp.mean(cent, axis=0, keepdims=True)
    xc = cent - adj
    g = jax.lax.dot_general(xc, xc, (((1,), (1,)), ((), ())),
                            preferred_element_type=jnp.float32)
    eye = jnp.where(
        jax.lax.broadcasted_iota(jnp.int32, (C, C), 0)
        == jax.lax.broadcasted_iota(jnp.int32, (C, C), 1), 1.0, 0.0)
    nrow = jnp.sum(g * eye, axis=0, keepdims=True)
    ncol = jnp.sum(g * eye, axis=1, keepdims=True)
    d = jnp.sqrt(jnp.clip(ncol + nrow - 2.0 * g, 1e-30, None))
    cl = jnp.clip((1.0 - eye) * MARGIN - d, 0.0, None)
    l2_ref[...] = jnp.reshape(jnp.sum(cl * cl), (1, 1))


def _final(cnt, a1, cent):
    return pl.pallas_call(
        _final_body,
        in_specs=[
            pl.BlockSpec((S, K * C), lambda: (0, 0)),
            pl.BlockSpec((S, K * C), lambda: (0, 0)),
            pl.BlockSpec((C, D), lambda: (0, 0)),
        ],
        out_specs=[
            pl.BlockSpec((1, 1), lambda: (0, 0)),
            pl.BlockSpec((1, 1), lambda: (0, 0)),
        ],
        out_shape=[
            jax.ShapeDtypeStruct((1, 1), jnp.float32),
            jax.ShapeDtypeStruct((1, 1), jnp.float32),
        ],
    )(cnt, a1, cent)


def kernel(fushed_features, input_data, centroids, gcn_w, bn_gamma, bn_beta,
           adj_mx_topk_index):
    sq = jnp.reshape(input_data, (B, S, T))
    w96 = jnp.reshape(gcn_w, (CT, T))
    gamma = jnp.reshape(bn_gamma, (S, 1))
    beta = jnp.reshape(bn_beta, (S, 1))

    st = _stats(fushed_features)
    q, sp, gidx = _dense(fushed_features, sq, st, gamma, beta, centroids,
                         w96, adj_mx_topk_index)

    r = _sc_gather(jnp.reshape(sp, (B * S, _ROWW)),
                   jnp.reshape(gidx, (_NTOT,)))
    r2 = jnp.reshape(r, (B, S, K * _ROWW))

    upd, cnt, a1, l3 = _reduce(q, adj_mx_topk_index, r2)
    l1, l2 = _final(cnt, a1, centroids)

    return (jnp.reshape(upd, (B, S, C, TO)),
            l1[0, 0], l2[0, 0], l3[0, 0])
